# Initial kernel scaffold; baseline (speedup 1.0000x reference)
import jax
import jax.numpy as jnp
from jax.experimental import pallas as pl


def kernel(x, ids, W1, b1, W2, b2, W3, b3):
    raise NotImplementedError("write your pallas kernel here")


# speedup vs baseline: 7.3451x; 7.3451x over previous
import jax
import jax.numpy as jnp
from jax.experimental import pallas as pl
from jax.experimental.pallas import tpu as pltpu

N = 32768
D = 512
H = 1024
NB = 16
BLK = 2048


def _fused_body(ids_ref, x_ref, w1_ref, b1_ref, w2_ref, b2_ref, w3_ref, b3_ref,
                out_ref, acc_ref, cnt_ref):
    i = pl.program_id(0)
    nsteps = pl.num_programs(0)

    @pl.when(i == 0)
    def _init():
        acc_ref[...] = jnp.zeros_like(acc_ref)
        cnt_ref[...] = jnp.zeros_like(cnt_ref)

    h = jnp.dot(x_ref[...], w1_ref[...], preferred_element_type=jnp.float32)
    h = jnp.maximum(h + b1_ref[...], 0.0)

    ids_blk = ids_ref[0, :]
    onehot = (ids_blk[None, :] ==
              jax.lax.broadcasted_iota(jnp.int32, (NB, BLK), 0)).astype(jnp.float32)
    acc_ref[...] += jnp.dot(onehot, h, preferred_element_type=jnp.float32)
    cnt_ref[...] += jnp.sum(onehot, axis=1, keepdims=True)

    @pl.when(i == nsteps - 1)
    def _finish():
        cnt = jnp.maximum(cnt_ref[:, :1], 1.0)
        s = jnp.dot(acc_ref[...], w2_ref[...], preferred_element_type=jnp.float32)
        agg = s / cnt + b2_ref[...]
        out_ref[...] = jnp.dot(agg, w3_ref[...], preferred_element_type=jnp.float32) + b3_ref[...]


def kernel(x, ids, W1, b1, W2, b2, W3, b3):
    inner_ids = ids[-1].astype(jnp.int32).reshape(1, N)
    grid = (N // BLK,)
    out = pl.pallas_call(
        _fused_body,
        grid=grid,
        in_specs=[
            pl.BlockSpec((1, BLK), lambda i: (0, i)),
            pl.BlockSpec((BLK, D), lambda i: (i, 0)),
            pl.BlockSpec((D, H), lambda i: (0, 0)),
            pl.BlockSpec((1, H), lambda i: (0, 0)),
            pl.BlockSpec((H, D), lambda i: (0, 0)),
            pl.BlockSpec((1, D), lambda i: (0, 0)),
            pl.BlockSpec((D, 128), lambda i: (0, 0)),
            pl.BlockSpec((1, 128), lambda i: (0, 0)),
        ],
        out_specs=pl.BlockSpec((NB, 128), lambda i: (0, 0)),
        out_shape=jax.ShapeDtypeStruct((NB, 128), jnp.float32),
        scratch_shapes=[
            pltpu.VMEM((NB, H), jnp.float32),
            pltpu.VMEM((NB, 128), jnp.float32),
        ],
        compiler_params=pltpu.CompilerParams(
            dimension_semantics=("arbitrary",),
        ),
    )(inner_ids, x, W1, b1.reshape(1, H), W2, b2.reshape(1, D), W3, b3.reshape(1, 128))
    return out
